# Initial kernel scaffold; baseline (speedup 1.0000x reference)
import jax
import jax.numpy as jnp
from jax.experimental import pallas as pl


def kernel(dec_outs, labels):
    raise NotImplementedError("write your pallas kernel here")


# speedup vs baseline: 4.7092x; 4.7092x over previous
import math

import jax
import jax.numpy as jnp
from jax.experimental import pallas as pl
from jax.experimental.pallas import tpu as pltpu

_LABEL_SMOOTHING = 0.1
_CONFIDENCE = 1.0 - _LABEL_SMOOTHING

_ROWS_PER_BLOCK = 64


def _loss_kernel(fill_term, conf_term, fill, conf, x_ref, lab_ref, o_ref):
    x = x_ref[...]
    lab = lab_ref[...]
    m = jnp.max(x, axis=1, keepdims=True)
    s = jnp.sum(jnp.exp(x - m), axis=1, keepdims=True)
    lse = m + jnp.log(s)
    scores = x - lse
    cols = jax.lax.broadcasted_iota(jnp.int32, x.shape, 1)
    is_lab = cols == lab
    o_ref[...] = jnp.where(is_lab, conf_term - conf * scores,
                           fill_term - fill * scores)


def kernel(dec_outs, labels):
    n, v = dec_outs.shape
    fill = _LABEL_SMOOTHING / (v - 1)
    fill_term = fill * math.log(fill)
    conf = _CONFIDENCE
    conf_term = conf * math.log(conf)

    r = _ROWS_PER_BLOCK
    grid = (n // r,)
    lab2d = labels.reshape(n, 1)

    return pl.pallas_call(
        lambda x_ref, lab_ref, o_ref: _loss_kernel(
            fill_term, conf_term, fill, conf, x_ref, lab_ref, o_ref),
        grid=grid,
        in_specs=[
            pl.BlockSpec((r, v), lambda i: (i, 0)),
            pl.BlockSpec((r, 1), lambda i: (i, 0)),
        ],
        out_specs=pl.BlockSpec((r, v), lambda i: (i, 0)),
        out_shape=jax.ShapeDtypeStruct((n, v), dec_outs.dtype),
        compiler_params=pltpu.CompilerParams(
            dimension_semantics=("arbitrary",),
        ),
    )(dec_outs, lab2d)


# speedup vs baseline: 5.0422x; 1.0707x over previous
import math

import jax
import jax.numpy as jnp
from jax.experimental import pallas as pl
from jax.experimental.pallas import tpu as pltpu

_LABEL_SMOOTHING = 0.1
_CONFIDENCE = 1.0 - _LABEL_SMOOTHING

_ROWS_PER_BLOCK = 64


def _loss_kernel(fill_term, conf_term, fill, conf, x_ref, lab_ref, o_ref):
    x = x_ref[...]
    lab = lab_ref[...]
    m = jnp.max(x, axis=1, keepdims=True)
    s = jnp.sum(jnp.exp(x - m), axis=1, keepdims=True)
    lse = m + jnp.log(s)
    fill_row = fill_term + fill * lse
    conf_row = conf_term + conf * lse
    cols = jax.lax.broadcasted_iota(jnp.int32, x.shape, 1)
    eq = cols == lab
    coef = jnp.where(eq, conf, fill)
    const = jnp.where(eq, conf_row, fill_row)
    o_ref[...] = const - coef * x


def kernel(dec_outs, labels):
    n, v = dec_outs.shape
    fill = _LABEL_SMOOTHING / (v - 1)
    fill_term = fill * math.log(fill)
    conf = _CONFIDENCE
    conf_term = conf * math.log(conf)

    r = _ROWS_PER_BLOCK
    grid = (n // r,)
    lab2d = labels.reshape(n, 1)

    return pl.pallas_call(
        lambda x_ref, lab_ref, o_ref: _loss_kernel(
            fill_term, conf_term, fill, conf, x_ref, lab_ref, o_ref),
        grid=grid,
        in_specs=[
            pl.BlockSpec((r, v), lambda i: (i, 0)),
            pl.BlockSpec((r, 1), lambda i: (i, 0)),
        ],
        out_specs=pl.BlockSpec((r, v), lambda i: (i, 0)),
        out_shape=jax.ShapeDtypeStruct((n, v), dec_outs.dtype),
        compiler_params=pltpu.CompilerParams(
            dimension_semantics=("parallel",),
        ),
    )(dec_outs, lab2d)
